# baseline (speedup 1.0000x reference)
import functools

import jax
import jax.numpy as jnp
from jax import lax
from jax.experimental import pallas as pl
from jax.experimental.pallas import tpu as pltpu
from jax.experimental.pallas import tpu_sc as plsc

N = 10000
NP = 10240
FDIM = 128
E = 320000
NCLS = 10
NC, NS = 2, 16
NW = NC * NS
EPW = E // NW
CH = 125
CPW = EPW // CH
DCH = 125
DCPW = EPW // DCH
ROWS_PT = NP // NS
DEG_PT = NP // NS
B = 1024
BPW = B // NW

_MESH = plsc.VectorSubcoreMesh(
    core_axis_name="c", subcore_axis_name="s", num_cores=NC, num_subcores=NS)


def _degree_body(dst2d, z640, ones_hbm, out,
                 didx_all, ones_v, zbuf, spdeg, sem):
    cid = lax.axis_index("c")
    sid = lax.axis_index("s")
    wid = sid * NC + cid
    pltpu.sync_copy(ones_hbm, ones_v)
    pltpu.sync_copy(z640, zbuf)
    pltpu.sync_copy(dst2d.at[pl.ds(wid * DCPW, DCPW)], didx_all)
    pltpu.sync_copy(zbuf, spdeg.at[pl.ds(sid * DEG_PT, DEG_PT)])
    plsc.subcore_barrier()

    def _grp(g, carry):
        descs = [
            pltpu.async_copy(ones_v, spdeg.at[didx_all.at[g * 8 + t]], sem,
                             add=True)
            for t in range(8)
        ]
        for d in descs:
            d.wait()
        return carry

    lax.fori_loop(0, DCPW // 8, _grp, 0)

    plsc.subcore_barrier()
    pltpu.sync_copy(spdeg.at[pl.ds(sid * DEG_PT, DEG_PT)],
                    out.at[pl.ds(cid * NP + sid * DEG_PT, DEG_PT)])


_degree = pl.kernel(
    _degree_body,
    out_type=jax.ShapeDtypeStruct((2 * NP,), jnp.float32),
    mesh=_MESH,
    scratch_types=[
        pltpu.VMEM((DCPW, DCH), jnp.int32),
        pltpu.VMEM((DCH,), jnp.float32),
        pltpu.VMEM((DEG_PT,), jnp.float32),
        pltpu.VMEM_SHARED((NP,), jnp.float32),
        pltpu.SemaphoreType.DMA,
    ],
)


GRP = 8
NG = CPW // GRP


def _propagate_body(src2d, dst2d, h, z80, out,
                    si0, si1, di0, di1, r0, r1,
                    spacc, ig0, ig1, gs0, gs1, ss0, ss1):
    cid = lax.axis_index("c")
    sid = lax.axis_index("s")
    wid = sid * NC + cid
    si = [si0, si1]
    di = [di0, di1]
    rows = [r0, r1]
    isem = [ig0, ig1]
    gsem = [gs0, gs1]
    ssem = [ss0, ss1]
    gbase = wid * CPW

    pltpu.sync_copy(z80, r0.at[pl.ds(0, 80)])
    base_r = sid * ROWS_PT
    for k in range(ROWS_PT // 80):
        pltpu.sync_copy(r0.at[pl.ds(0, 80)],
                        spacc.at[pl.ds(base_r + k * 80, 80)])
    plsc.subcore_barrier()

    def idx_issue(g, slot):
        pltpu.async_copy(src2d.at[pl.ds(gbase + g * GRP, GRP)], si[slot],
                         isem[slot])
        pltpu.async_copy(dst2d.at[pl.ds(gbase + g * GRP, GRP)], di[slot],
                         isem[slot])

    def idx_wait(g, slot):
        pltpu.make_async_copy(src2d.at[pl.ds(gbase + g * GRP, GRP)],
                              si[slot], isem[slot]).wait()
        pltpu.make_async_copy(dst2d.at[pl.ds(gbase + g * GRP, GRP)],
                              di[slot], isem[slot]).wait()

    def gather(s, k, b):
        pltpu.async_copy(h.at[si[s].at[k]], rows[b], gsem[b])

    def gather_wait(s, k, b):
        pltpu.make_async_copy(h.at[si[s].at[k]], rows[b], gsem[b]).wait()

    def scat(s, k, b):
        pltpu.async_copy(rows[b], spacc.at[di[s].at[k]], ssem[b], add=True)

    def scat_wait(s, k, b):
        pltpu.make_async_copy(rows[b], spacc.at[di[s].at[k]], ssem[b]).wait()

    def group_chunks(g, s, last_group):
        for k in range(GRP):
            b = k % 2
            gather_wait(s, k, b)
            scat(s, k, b)
            scat_wait(s, k, b)
            if k == 6 and not last_group:
                idx_wait(g + 1, 1 - s)
            if k < 6:
                gather(s, k + 2, b)
            elif not last_group:
                gather(1 - s, k - 6, b)

    idx_issue(0, 0)
    idx_issue(1, 1)
    idx_wait(0, 0)
    gather(0, 0, 0)
    gather(0, 1, 1)

    def pair(gp, carry):
        g = 2 * gp
        group_chunks(g, 0, False)
        idx_issue(g + 2, 0)
        group_chunks(g + 1, 1, False)
        idx_issue(g + 3, 1)
        return carry

    lax.fori_loop(0, (NG - 2) // 2, pair, 0)
    group_chunks(NG - 2, 0, False)
    group_chunks(NG - 1, 1, True)

    plsc.subcore_barrier()
    pltpu.sync_copy(spacc.at[pl.ds(base_r, ROWS_PT)],
                    out.at[pl.ds(cid * NP + base_r, ROWS_PT)])


_propagate = pl.kernel(
    _propagate_body,
    out_type=jax.ShapeDtypeStruct((2 * NP, FDIM), jnp.float32),
    mesh=_MESH,
    scratch_types=[
        pltpu.VMEM((GRP, CH), jnp.int32),
        pltpu.VMEM((GRP, CH), jnp.int32),
        pltpu.VMEM((GRP, CH), jnp.int32),
        pltpu.VMEM((GRP, CH), jnp.int32),
        pltpu.VMEM((CH, FDIM), jnp.float32),
        pltpu.VMEM((CH, FDIM), jnp.float32),
        pltpu.VMEM_SHARED((NP, FDIM), jnp.float32),
        pltpu.SemaphoreType.DMA,
        pltpu.SemaphoreType.DMA,
        pltpu.SemaphoreType.DMA,
        pltpu.SemaphoreType.DMA,
        pltpu.SemaphoreType.DMA,
        pltpu.SemaphoreType.DMA,
    ],
)


def _select_body(bi, p, hp, deg, oa, ob, oh, oda, odb,
                 bidx, bidxb, ra, rb, rh, dsa, dsb, sa, sb, sh, sd, sd2):
    cid = lax.axis_index("c")
    sid = lax.axis_index("s")
    wid = sid * NC + cid
    base = wid * BPW
    sl = pl.ds(base, BPW)
    pltpu.sync_copy(bi.at[sl], bidx)
    for k in range(BPW // 16):
        ks = pl.ds(k * 16, 16)
        bidxb[ks] = bidx[ks] + NP
    pltpu.async_copy(p.at[bidx], ra, sa)
    pltpu.async_copy(p.at[bidxb], rb, sb)
    pltpu.async_copy(hp.at[bidx], rh, sh)
    pltpu.async_copy(deg.at[bidx], dsa, sd)
    pltpu.async_copy(deg.at[bidxb], dsb, sd2)
    pltpu.make_async_copy(p.at[bidx], ra, sa).wait()
    pltpu.sync_copy(ra, oa.at[sl])
    pltpu.make_async_copy(p.at[bidxb], rb, sb).wait()
    pltpu.sync_copy(rb, ob.at[sl])
    pltpu.make_async_copy(hp.at[bidx], rh, sh).wait()
    pltpu.sync_copy(rh, oh.at[sl])
    pltpu.make_async_copy(deg.at[bidx], dsa, sd).wait()
    pltpu.sync_copy(dsa, oda.at[sl])
    pltpu.make_async_copy(deg.at[bidxb], dsb, sd2).wait()
    pltpu.sync_copy(dsb, odb.at[sl])


_select = pl.kernel(
    _select_body,
    out_type=[jax.ShapeDtypeStruct((B, FDIM), jnp.float32),
              jax.ShapeDtypeStruct((B, FDIM), jnp.float32),
              jax.ShapeDtypeStruct((B, FDIM), jnp.float32),
              jax.ShapeDtypeStruct((B,), jnp.float32),
              jax.ShapeDtypeStruct((B,), jnp.float32)],
    mesh=_MESH,
    scratch_types=[
        pltpu.VMEM((BPW,), jnp.int32),
        pltpu.VMEM((BPW,), jnp.int32),
        pltpu.VMEM((BPW, FDIM), jnp.float32),
        pltpu.VMEM((BPW, FDIM), jnp.float32),
        pltpu.VMEM((BPW, FDIM), jnp.float32),
        pltpu.VMEM((BPW,), jnp.float32),
        pltpu.VMEM((BPW,), jnp.float32),
        pltpu.SemaphoreType.DMA,
        pltpu.SemaphoreType.DMA,
        pltpu.SemaphoreType.DMA,
        pltpu.SemaphoreType.DMA,
        pltpu.SemaphoreType.DMA,
    ],
)


def _mm_scale_kernel(x, w, da, db, o):
    dis = lax.rsqrt(da[...] + db[...] + 1.0)
    o[...] = dis * jnp.dot(x[...], w[...],
                           preferred_element_type=jnp.float32)


def _layer_kernel(pa, pb, hp, da, db, b, w, o):
    dis = lax.rsqrt(da[...] + db[...] + 1.0)
    a = jnp.maximum(dis * (pa[...] + pb[...] + hp[...]) + b[...], 0.0)
    o[...] = dis * jnp.dot(a, w[...],
                           preferred_element_type=jnp.float32)


def _head_kernel(pa, pb, ph, da, db, b2, wl, bl, logp, outp, feats):
    dis = lax.rsqrt(da[...] + db[...] + 1.0)
    f = jnp.maximum(dis * (pa[...] + pb[...] + ph[...]) + b2[...], 0.0)
    feats[...] = f
    z = jnp.maximum(
        jnp.dot(f, wl[...], preferred_element_type=jnp.float32) + bl[...], 0.0)
    outp[...] = z
    m = jnp.max(z, axis=1, keepdims=True)
    logp[...] = (z - m) - jnp.log(
        jnp.sum(jnp.exp(z - m), axis=1, keepdims=True))


_RB = 640


def _call_mm_scale(x, w, deg2d):
    grid = (NP // _RB,)
    return pl.pallas_call(
        _mm_scale_kernel,
        grid=grid,
        in_specs=[
            pl.BlockSpec((_RB, FDIM), lambda i: (i, 0)),
            pl.BlockSpec((FDIM, FDIM), lambda i: (0, 0)),
            pl.BlockSpec((_RB, 1), lambda i: (i, 0)),
            pl.BlockSpec((_RB, 1), lambda i: (i + NP // _RB, 0)),
        ],
        out_specs=pl.BlockSpec((_RB, FDIM), lambda i: (i, 0)),
        out_shape=jax.ShapeDtypeStruct((NP, FDIM), jnp.float32),
    )(x, w, deg2d, deg2d)


def _call_layer(p, hp, deg2d, b, w):
    grid = (NP // _RB,)
    return pl.pallas_call(
        _layer_kernel,
        grid=grid,
        in_specs=[
            pl.BlockSpec((_RB, FDIM), lambda i: (i, 0)),
            pl.BlockSpec((_RB, FDIM), lambda i: (i + NP // _RB, 0)),
            pl.BlockSpec((_RB, FDIM), lambda i: (i, 0)),
            pl.BlockSpec((_RB, 1), lambda i: (i, 0)),
            pl.BlockSpec((_RB, 1), lambda i: (i + NP // _RB, 0)),
            pl.BlockSpec((1, FDIM), lambda i: (0, 0)),
            pl.BlockSpec((FDIM, FDIM), lambda i: (0, 0)),
        ],
        out_specs=pl.BlockSpec((_RB, FDIM), lambda i: (i, 0)),
        out_shape=jax.ShapeDtypeStruct((NP, FDIM), jnp.float32),
    )(p, p, hp, deg2d, deg2d, b, w)


def _call_head(pa, pb, ph, da, db, b2, wl, bl):
    return pl.pallas_call(
        _head_kernel,
        out_shape=[jax.ShapeDtypeStruct((B, NCLS), jnp.float32),
                   jax.ShapeDtypeStruct((B, NCLS), jnp.float32),
                   jax.ShapeDtypeStruct((B, FDIM), jnp.float32)],
    )(pa, pb, ph, da, db, b2, wl, bl)


def kernel(x, edge_index, batch_index, W1, b1, W2, b2, Wlin, blin):
    ei = edge_index.astype(jnp.int32)
    src2d = ei[0].reshape(E // CH, CH)
    dst2d = ei[1].reshape(E // CH, CH)
    dstd = ei[1].reshape(E // DCH, DCH)
    bi = batch_index.astype(jnp.int32)
    z80 = jnp.zeros((80, FDIM), jnp.float32)
    z640 = jnp.zeros((DEG_PT,), jnp.float32)
    ones_v = jnp.ones((DCH,), jnp.float32)
    xp = jnp.zeros((NP, FDIM), jnp.float32).at[:N].set(x)

    deg2 = _degree(dstd, z640, ones_v)
    deg2d = deg2.reshape(2 * NP, 1)

    h1p = _call_mm_scale(xp, W1, deg2d)
    p1 = _propagate(src2d, dst2d, h1p, z80)
    h2p = _call_layer(p1, h1p, deg2d, b1.reshape(1, FDIM), W2)
    p2 = _propagate(src2d, dst2d, h2p, z80)

    sa, sb, sh, dga, dgb = _select(bi, p2, h2p, deg2)
    logp, outp, feats = _call_head(sa, sb, sh, dga.reshape(B, 1),
                                   dgb.reshape(B, 1),
                                   b2.reshape(1, FDIM), Wlin,
                                   blin.reshape(1, NCLS))
    return (logp, outp, feats)


# speedup vs baseline: 1.0127x; 1.0127x over previous
import functools

import jax
import jax.numpy as jnp
from jax import lax
from jax.experimental import pallas as pl
from jax.experimental.pallas import tpu as pltpu
from jax.experimental.pallas import tpu_sc as plsc

N = 10000
NP = 10240
FDIM = 128
E = 320000
NCLS = 10
NC, NS = 2, 16
NW = NC * NS
EPW = E // NW
CH = 125
CPW = EPW // CH
DCH = 125
DCPW = EPW // DCH
ROWS_PT = NP // NS
DEG_PT = NP // NS
B = 1024
BPW = B // NW

_MESH = plsc.VectorSubcoreMesh(
    core_axis_name="c", subcore_axis_name="s", num_cores=NC, num_subcores=NS)


def _degree_body(dst2d, z640, ones_hbm, out,
                 didx_all, ones_v, zbuf, spdeg, sem):
    cid = lax.axis_index("c")
    sid = lax.axis_index("s")
    wid = sid * NC + cid
    pltpu.sync_copy(ones_hbm, ones_v)
    pltpu.sync_copy(z640, zbuf)
    pltpu.sync_copy(dst2d.at[pl.ds(wid * DCPW, DCPW)], didx_all)
    pltpu.sync_copy(zbuf, spdeg.at[pl.ds(sid * DEG_PT, DEG_PT)])
    plsc.subcore_barrier()

    def _grp(g, carry):
        descs = [
            pltpu.async_copy(ones_v, spdeg.at[didx_all.at[g * 8 + t]], sem,
                             add=True)
            for t in range(8)
        ]
        for d in descs:
            d.wait()
        return carry

    lax.fori_loop(0, DCPW // 8, _grp, 0)

    plsc.subcore_barrier()
    pltpu.sync_copy(spdeg.at[pl.ds(sid * DEG_PT, DEG_PT)],
                    out.at[pl.ds(cid * NP + sid * DEG_PT, DEG_PT)])


_degree = pl.kernel(
    _degree_body,
    out_type=jax.ShapeDtypeStruct((2 * NP,), jnp.float32),
    mesh=_MESH,
    scratch_types=[
        pltpu.VMEM((DCPW, DCH), jnp.int32),
        pltpu.VMEM((DCH,), jnp.float32),
        pltpu.VMEM((DEG_PT,), jnp.float32),
        pltpu.VMEM_SHARED((NP,), jnp.float32),
        pltpu.SemaphoreType.DMA,
    ],
)


GRP = 8
NG = CPW // GRP


def _propagate_body(src2d, dst2d, h, z80, out,
                    si0, si1, di0, di1, r0, r1,
                    spacc, ig0, ig1, gs0, gs1, ss0, ss1):
    cid = lax.axis_index("c")
    sid = lax.axis_index("s")
    wid = sid * NC + cid
    si = [si0, si1]
    di = [di0, di1]
    rows = [r0, r1]
    isem = [ig0, ig1]
    gsem = [gs0, gs1]
    ssem = [ss0, ss1]
    gbase = wid * CPW

    pltpu.sync_copy(z80, r0.at[pl.ds(0, 80)])
    base_r = sid * ROWS_PT
    for k in range(ROWS_PT // 80):
        pltpu.sync_copy(r0.at[pl.ds(0, 80)],
                        spacc.at[pl.ds(base_r + k * 80, 80)])
    plsc.subcore_barrier()

    def idx_issue(g, slot):
        pltpu.async_copy(src2d.at[pl.ds(gbase + g * GRP, GRP)], si[slot],
                         isem[slot])
        pltpu.async_copy(dst2d.at[pl.ds(gbase + g * GRP, GRP)], di[slot],
                         isem[slot])

    def idx_wait(g, slot):
        pltpu.make_async_copy(src2d.at[pl.ds(gbase + g * GRP, GRP)],
                              si[slot], isem[slot]).wait()
        pltpu.make_async_copy(dst2d.at[pl.ds(gbase + g * GRP, GRP)],
                              di[slot], isem[slot]).wait()

    def gather(s, k, b):
        pltpu.async_copy(h.at[si[s].at[k]], rows[b], gsem[b])

    def gather_wait(s, k, b):
        pltpu.make_async_copy(h.at[si[s].at[k]], rows[b], gsem[b]).wait()

    def scat(s, k, b):
        pltpu.async_copy(rows[b], spacc.at[di[s].at[k]], ssem[b], add=True)

    def scat_wait(s, k, b):
        pltpu.make_async_copy(rows[b], spacc.at[di[s].at[k]], ssem[b]).wait()

    def group_chunks(g, s, last_group):
        for k in range(GRP):
            b = k % 2
            gather_wait(s, k, b)
            scat(s, k, b)
            scat_wait(s, k, b)
            if k == 6 and not last_group:
                idx_wait(g + 1, 1 - s)
            if k < 6:
                gather(s, k + 2, b)
            elif not last_group:
                gather(1 - s, k - 6, b)

    idx_issue(0, 0)
    idx_issue(1, 1)
    idx_wait(0, 0)
    gather(0, 0, 0)
    gather(0, 1, 1)

    def pair(gp, carry):
        g = 2 * gp
        group_chunks(g, 0, False)
        idx_issue(g + 2, 0)
        group_chunks(g + 1, 1, False)
        idx_issue(g + 3, 1)
        return carry

    lax.fori_loop(0, (NG - 2) // 2, pair, 0)
    group_chunks(NG - 2, 0, False)
    group_chunks(NG - 1, 1, True)

    plsc.subcore_barrier()
    pltpu.sync_copy(spacc.at[pl.ds(base_r, ROWS_PT)],
                    out.at[pl.ds(cid * NP + base_r, ROWS_PT)])


_propagate = pl.kernel(
    _propagate_body,
    out_type=jax.ShapeDtypeStruct((2 * NP, FDIM), jnp.float32),
    mesh=_MESH,
    scratch_types=[
        pltpu.VMEM((GRP, CH), jnp.int32),
        pltpu.VMEM((GRP, CH), jnp.int32),
        pltpu.VMEM((GRP, CH), jnp.int32),
        pltpu.VMEM((GRP, CH), jnp.int32),
        pltpu.VMEM((CH, FDIM), jnp.float32),
        pltpu.VMEM((CH, FDIM), jnp.float32),
        pltpu.VMEM_SHARED((NP, FDIM), jnp.float32),
        pltpu.SemaphoreType.DMA,
        pltpu.SemaphoreType.DMA,
        pltpu.SemaphoreType.DMA,
        pltpu.SemaphoreType.DMA,
        pltpu.SemaphoreType.DMA,
        pltpu.SemaphoreType.DMA,
    ],
)


def _select_body(bi, p, hp, dis, oa, ob, oh, od,
                 bidx, bidxb, ra, rb, rh, dsel, sa, sb, sh, sd):
    cid = lax.axis_index("c")
    sid = lax.axis_index("s")
    wid = sid * NC + cid
    base = wid * BPW
    sl = pl.ds(base, BPW)
    pltpu.sync_copy(bi.at[sl], bidx)
    for k in range(BPW // 16):
        ks = pl.ds(k * 16, 16)
        bidxb[ks] = bidx[ks] + NP
    pltpu.async_copy(p.at[bidx], ra, sa)
    pltpu.async_copy(p.at[bidxb], rb, sb)
    pltpu.async_copy(hp.at[bidx], rh, sh)
    pltpu.async_copy(dis.at[bidx], dsel, sd)
    pltpu.make_async_copy(p.at[bidx], ra, sa).wait()
    pltpu.sync_copy(ra, oa.at[sl])
    pltpu.make_async_copy(p.at[bidxb], rb, sb).wait()
    pltpu.sync_copy(rb, ob.at[sl])
    pltpu.make_async_copy(hp.at[bidx], rh, sh).wait()
    pltpu.sync_copy(rh, oh.at[sl])
    pltpu.make_async_copy(dis.at[bidx], dsel, sd).wait()
    pltpu.sync_copy(dsel, od.at[sl])


_select = pl.kernel(
    _select_body,
    out_type=[jax.ShapeDtypeStruct((B, FDIM), jnp.float32),
              jax.ShapeDtypeStruct((B, FDIM), jnp.float32),
              jax.ShapeDtypeStruct((B, FDIM), jnp.float32),
              jax.ShapeDtypeStruct((B,), jnp.float32)],
    mesh=_MESH,
    scratch_types=[
        pltpu.VMEM((BPW,), jnp.int32),
        pltpu.VMEM((BPW,), jnp.int32),
        pltpu.VMEM((BPW, FDIM), jnp.float32),
        pltpu.VMEM((BPW, FDIM), jnp.float32),
        pltpu.VMEM((BPW, FDIM), jnp.float32),
        pltpu.VMEM((BPW,), jnp.float32),
        pltpu.SemaphoreType.DMA,
        pltpu.SemaphoreType.DMA,
        pltpu.SemaphoreType.DMA,
        pltpu.SemaphoreType.DMA,
    ],
)


def _dis_kernel(d, o):
    dd = d[...]
    o[...] = lax.rsqrt(dd[:NP // 128] + dd[NP // 128:] + 1.0)


def _mm_scale_kernel(x, w, dis, o):
    o[...] = dis[...] * jnp.dot(x[...], w[...],
                                preferred_element_type=jnp.float32)


def _layer_kernel(pa, pb, hp, dis, b, w, o):
    a = jnp.maximum(dis[...] * (pa[...] + pb[...] + hp[...]) + b[...], 0.0)
    o[...] = dis[...] * jnp.dot(a, w[...],
                                preferred_element_type=jnp.float32)


def _head_kernel(pa, pb, ph, dis, b2, wl, bl, logp, outp, feats):
    f = jnp.maximum(dis[...] * (pa[...] + pb[...] + ph[...]) + b2[...], 0.0)
    feats[...] = f
    z = jnp.maximum(
        jnp.dot(f, wl[...], preferred_element_type=jnp.float32) + bl[...], 0.0)
    outp[...] = z
    m = jnp.max(z, axis=1, keepdims=True)
    logp[...] = (z - m) - jnp.log(
        jnp.sum(jnp.exp(z - m), axis=1, keepdims=True))


_RB = 640


def _call_dis(d):
    return pl.pallas_call(
        _dis_kernel,
        out_shape=jax.ShapeDtypeStruct((NP // 128, 128), jnp.float32),
    )(d)


def _call_mm_scale(x, w, dis2d):
    grid = (NP // _RB,)
    return pl.pallas_call(
        _mm_scale_kernel,
        grid=grid,
        in_specs=[
            pl.BlockSpec((_RB, FDIM), lambda i: (i, 0)),
            pl.BlockSpec((FDIM, FDIM), lambda i: (0, 0)),
            pl.BlockSpec((_RB, 1), lambda i: (i, 0)),
        ],
        out_specs=pl.BlockSpec((_RB, FDIM), lambda i: (i, 0)),
        out_shape=jax.ShapeDtypeStruct((NP, FDIM), jnp.float32),
    )(x, w, dis2d)


def _call_layer(p, hp, dis2d, b, w):
    grid = (NP // _RB,)
    return pl.pallas_call(
        _layer_kernel,
        grid=grid,
        in_specs=[
            pl.BlockSpec((_RB, FDIM), lambda i: (i, 0)),
            pl.BlockSpec((_RB, FDIM), lambda i: (i + NP // _RB, 0)),
            pl.BlockSpec((_RB, FDIM), lambda i: (i, 0)),
            pl.BlockSpec((_RB, 1), lambda i: (i, 0)),
            pl.BlockSpec((1, FDIM), lambda i: (0, 0)),
            pl.BlockSpec((FDIM, FDIM), lambda i: (0, 0)),
        ],
        out_specs=pl.BlockSpec((_RB, FDIM), lambda i: (i, 0)),
        out_shape=jax.ShapeDtypeStruct((NP, FDIM), jnp.float32),
    )(p, p, hp, dis2d, b, w)


def _call_head(pa, pb, ph, dis2d, b2, wl, bl):
    return pl.pallas_call(
        _head_kernel,
        out_shape=[jax.ShapeDtypeStruct((B, NCLS), jnp.float32),
                   jax.ShapeDtypeStruct((B, NCLS), jnp.float32),
                   jax.ShapeDtypeStruct((B, FDIM), jnp.float32)],
    )(pa, pb, ph, dis2d, b2, wl, bl)


def kernel(x, edge_index, batch_index, W1, b1, W2, b2, Wlin, blin):
    ei = edge_index.astype(jnp.int32)
    src2d = ei[0].reshape(E // CH, CH)
    dst2d = ei[1].reshape(E // CH, CH)
    dstd = ei[1].reshape(E // DCH, DCH)
    bi = batch_index.astype(jnp.int32)
    z80 = jnp.zeros((80, FDIM), jnp.float32)
    z640 = jnp.zeros((DEG_PT,), jnp.float32)
    ones_v = jnp.ones((DCH,), jnp.float32)
    xp = jnp.zeros((NP, FDIM), jnp.float32).at[:N].set(x)

    deg2 = _degree(dstd, z640, ones_v)
    dis_flat = _call_dis(deg2.reshape(2 * NP // 128, 128)).reshape(NP)
    dis2d = dis_flat.reshape(NP, 1)

    h1p = _call_mm_scale(xp, W1, dis2d)
    p1 = _propagate(src2d, dst2d, h1p, z80)
    h2p = _call_layer(p1, h1p, dis2d, b1.reshape(1, FDIM), W2)
    p2 = _propagate(src2d, dst2d, h2p, z80)

    sa, sb, sh, dsel = _select(bi, p2, h2p, dis_flat)
    logp, outp, feats = _call_head(sa, sb, sh, dsel.reshape(B, 1),
                                   b2.reshape(1, FDIM), Wlin,
                                   blin.reshape(1, NCLS))
    return (logp, outp, feats)
